# baseline (speedup 1.0000x reference)
import jax
import jax.numpy as jnp
from jax import lax
from jax.experimental import pallas as pl
from jax.experimental.pallas import tpu as pltpu
from jax.experimental.pallas import tpu_sc as plsc

_NC, _NS = 2, 16
_NW = _NC * _NS


def _proj_body(obj_ref, ws_ref, bs_ref, wo_ref, bo_ref, s_ref, o_ref):
    x = obj_ref[...]
    s_ref[...] = jnp.dot(x, ws_ref[...],
                         preferred_element_type=jnp.float32) + bs_ref[...]
    o_ref[...] = jnp.dot(x, wo_ref[...],
                         preferred_element_type=jnp.float32) + bo_ref[...]


def _project(obj, Ws, bs, Wo, bo):
    n, d = obj.shape
    out = jax.ShapeDtypeStruct((n, d), jnp.float32)
    return pl.pallas_call(_proj_body, out_shape=(out, out))(
        obj, Ws, bs.reshape(1, d), Wo, bo.reshape(1, d))


def _edge_product(s, o, u, iarr, jarr):
    n, d = s.shape
    e = u.shape[0]
    epw = e // _NW
    ch = 32
    chn = epw // ch

    def body(s_hbm, o_hbm, u_hbm, i_hbm, j_hbm, t_hbm,
             ivall, jvall, ivc, jvc, sv, ov, uv, tv, sem1, sem2):
        wid = lax.axis_index("s") * _NC + lax.axis_index("c")
        ebase = wid * epw
        pltpu.sync_copy(i_hbm.at[pl.ds(ebase, epw)], ivall)
        pltpu.sync_copy(j_hbm.at[pl.ds(ebase, epw)], jvall)

        def chunk(it, carry):
            base = it * ch
            for q in range(ch // 16):
                ivc[pl.ds(q * 16, 16)] = ivall[pl.ds(base + q * 16, 16)]
                jvc[pl.ds(q * 16, 16)] = jvall[pl.ds(base + q * 16, 16)]
            cp1 = pltpu.async_copy(s_hbm.at[ivc], sv, sem1)
            cp2 = pltpu.async_copy(o_hbm.at[jvc], ov, sem2)
            pltpu.sync_copy(u_hbm.at[pl.ds(ebase + base, ch)], uv)
            cp1.wait()
            cp2.wait()

            def row(r, c2):
                for q in range(d // 16):
                    sl = pl.ds(q * 16, 16)
                    tv[r, sl] = sv[r, sl] * ov[r, sl] * uv[r, sl]
                return c2

            lax.fori_loop(0, ch, row, 0)
            pltpu.sync_copy(tv, t_hbm.at[pl.ds(ebase + base, ch)])
            return carry

        lax.fori_loop(0, chn, chunk, 0)

    mesh = plsc.VectorSubcoreMesh(core_axis_name="c", subcore_axis_name="s")
    fn = pl.kernel(
        body,
        out_type=jax.ShapeDtypeStruct((e, d), jnp.float32),
        mesh=mesh,
        compiler_params=pltpu.CompilerParams(use_tc_tiling_on_sc=False),
        scratch_types=[
            pltpu.VMEM((epw,), jnp.int32),
            pltpu.VMEM((epw,), jnp.int32),
            pltpu.VMEM((ch,), jnp.int32),
            pltpu.VMEM((ch,), jnp.int32),
            pltpu.VMEM((ch, d), jnp.float32),
            pltpu.VMEM((ch, d), jnp.float32),
            pltpu.VMEM((ch, d), jnp.float32),
            pltpu.VMEM((ch, d), jnp.float32),
            pltpu.SemaphoreType.DMA,
            pltpu.SemaphoreType.DMA,
        ],
    )
    return fn(s, o, u, iarr, jarr)


def _attf_body(t_ref, ww_ref, bw_ref, out_ref):
    out_ref[...] = jnp.dot(t_ref[...], ww_ref[...],
                           preferred_element_type=jnp.float32) + bw_ref[...]


def _atten_feats(t, Ww, bw):
    e, d = t.shape
    p = Ww.shape[1]
    blk = 2048
    return pl.pallas_call(
        _attf_body,
        grid=(e // blk,),
        in_specs=[
            pl.BlockSpec((blk, d), lambda i: (i, 0)),
            pl.BlockSpec((d, p), lambda i: (0, 0)),
            pl.BlockSpec((1, p), lambda i: (0, 0)),
        ],
        out_specs=pl.BlockSpec((blk, p), lambda i: (i, 0)),
        out_shape=jax.ShapeDtypeStruct((e, p), jnp.float32),
    )(t, Ww, bw.reshape(1, p))


def _scatter_dense(attf, iarr, jarr, n):
    e, p = attf.shape
    rows = 64
    nblk_per_sc = n // rows // _NC
    sprows = rows * n
    ept = e // _NS
    zrows = 1024

    def body(attf_hbm, i_hbm, j_hbm, z_hbm, av, iv, jv, xv, zerov, spbuf):
        c = lax.axis_index("c")
        s = lax.axis_index("s")
        tb = s * ept
        pltpu.sync_copy(attf_hbm.at[pl.ds(tb, ept)], av)
        pltpu.sync_copy(i_hbm.at[pl.ds(tb, ept)], iv)
        pltpu.sync_copy(j_hbm.at[pl.ds(tb, ept)], jv)

        def zrow(r, carry):
            zerov[r, :] = jnp.zeros((16,), jnp.float32)
            return carry

        lax.fori_loop(0, zrows, zrow, 0)

        for bb in range(nblk_per_sc):
            b = c * nblk_per_sc + bb
            for q in range(sprows // _NS // zrows):
                pltpu.sync_copy(
                    zerov, spbuf.at[pl.ds(s * (sprows // _NS) + q * zrows,
                                          zrows)])
            plsc.subcore_barrier()

            def ixc(k, carry):
                for u2 in range(4):
                    sl = pl.ds((k * 4 + u2) * 16, 16)
                    iv16 = iv[sl]
                    jv16 = jv[sl]
                    valid = (iv16 >> 6) == b
                    loc = ((iv16 & (rows - 1)) << 10) | jv16
                    xv[sl] = jnp.where(valid, loc, sprows)
                return carry

            lax.fori_loop(0, ept // 64, ixc, 0)
            pltpu.sync_copy(av, spbuf.at[xv], add=True)
            plsc.subcore_barrier()
            pltpu.sync_copy(
                spbuf.at[pl.ds(s * (sprows // _NS), sprows // _NS)],
                z_hbm.at[pl.ds(b * sprows + s * (sprows // _NS),
                               sprows // _NS)])
            plsc.subcore_barrier()

    mesh = plsc.VectorSubcoreMesh(core_axis_name="c", subcore_axis_name="s")
    fn = pl.kernel(
        body,
        out_type=jax.ShapeDtypeStruct((n * n, p), jnp.float32),
        mesh=mesh,
        compiler_params=pltpu.CompilerParams(use_tc_tiling_on_sc=False),
        scratch_types=[
            pltpu.VMEM((ept, p), jnp.float32),
            pltpu.VMEM((ept,), jnp.int32),
            pltpu.VMEM((ept,), jnp.int32),
            pltpu.VMEM((ept,), jnp.int32),
            pltpu.VMEM((zrows, p), jnp.float32),
            pltpu.VMEM_SHARED((sprows + 8, p), jnp.float32),
        ],
    )
    return fn(attf, iarr, jarr)


def _softmax_rows(z2, n, p):
    npcols = z2.shape[1]
    bi = 16
    nch = npcols // 128

    def body(z_ref, o_ref):
        pid = pl.program_id(0)
        rowv = pid * bi + lax.broadcasted_iota(jnp.int32, (bi, 128), 0)
        colb = lax.broadcasted_iota(jnp.int32, (bi, 128), 1)

        def masked(k):
            x = z_ref[:, k * 128:(k + 1) * 128]
            j = (k * 128 + colb) >> 4
            return x - jnp.where(j == rowv, 1e4, 0.0).astype(jnp.float32)

        m = masked(0)
        for k in range(1, nch):
            m = jnp.maximum(m, masked(k))
        for sft in (16, 32, 64):
            m = jnp.maximum(
                m, jnp.concatenate([m[:, sft:], m[:, :sft]], axis=1))
        ssum = jnp.zeros((bi, 128), jnp.float32)
        for k in range(nch):
            ex = jnp.exp(masked(k) - m)
            ssum = ssum + ex
            o_ref[:, k * 128:(k + 1) * 128] = ex
        for sft in (16, 32, 64):
            ssum = ssum + jnp.concatenate(
                [ssum[:, sft:], ssum[:, :sft]], axis=1)
        r = 1.0 / ssum
        for k in range(nch):
            o_ref[:, k * 128:(k + 1) * 128] = (
                o_ref[:, k * 128:(k + 1) * 128] * r)

    return pl.pallas_call(
        body,
        grid=(n // bi,),
        in_specs=[pl.BlockSpec((bi, npcols), lambda i: (i, 0))],
        out_specs=pl.BlockSpec((bi, npcols), lambda i: (i, 0)),
        out_shape=jax.ShapeDtypeStruct((n, npcols), jnp.float32),
    )(z2)


def kernel(obj_feats, union_feats, pair_idxs, Ws, bs, Wo, bo, Ww, bw):
    n, d = obj_feats.shape
    p = Ww.shape[1]
    s, o = _project(obj_feats, Ws, bs, Wo, bo)
    iarr = pair_idxs[:, 0]
    jarr = pair_idxs[:, 1]
    t = _edge_product(s, o, union_feats, iarr, jarr)
    attf = _atten_feats(t, Ww, bw)
    z = _scatter_dense(attf, iarr, jarr, n)
    out = _softmax_rows(z.reshape(n, n * p), n, p)
    return out.reshape(n, n, p)
